# Initial kernel scaffold; baseline (speedup 1.0000x reference)
import jax
import jax.numpy as jnp
from jax.experimental import pallas as pl


def kernel(points, features, leaf_mask):
    raise NotImplementedError("write your pallas kernel here")


# speedup vs baseline: 116.1211x; 116.1211x over previous
import jax
import jax.numpy as jnp
from jax.experimental import pallas as pl
from jax.experimental.pallas import tpu as pltpu

_R = 0.05


def _lcn_body(pts_row_ref, pts_col_ref, pts_smem_ref, lmf_ref, lm_smem_ref,
              feat_ref, out_ref, idx_ref):
    N = pts_row_ref.shape[1]
    C_MAX = idx_ref.shape[1]

    out_ref[...] = feat_ref[...]

    lmf = lmf_ref[0:1, :]
    total_leaf = jnp.sum(lmf)
    enough = total_leaf >= 10.0

    def scan_body(i, carry):
        nleaf, ncand = carry
        is_leaf = lm_smem_ref[0, i]
        is_cand = (is_leaf == 1) & (nleaf % 10 == 0)

        @pl.when(is_cand)
        def _():
            idx_ref[0, ncand] = i

        return (nleaf + is_leaf, ncand + is_cand.astype(jnp.int32))

    _, ncand = jax.lax.fori_loop(
        0, N, scan_body, (jnp.int32(0), jnp.int32(0)), unroll=False)
    ncand = jnp.where(enough, ncand, 0)

    def cand_body(c, carry):
        @pl.when(c < ncand)
        def _():
            i = idx_ref[0, c]
            cx = pts_smem_ref[0, i]
            cy = pts_smem_ref[1, i]
            cz = pts_smem_ref[2, i]

            dx = pts_row_ref[0:1, :] - cx
            dy = pts_row_ref[1:2, :] - cy
            dz = pts_row_ref[2:3, :] - cz
            d = jnp.sqrt(dx * dx + dy * dy + dz * dz)

            near = (d < _R).astype(jnp.float32)
            ann = ((d > _R) & (d < 2.0 * _R)).astype(jnp.float32)
            nearby_leaf = jnp.sum(near * lmf)
            w = ann * lmf
            cnt = jnp.sum(w)
            apply_update = (nearby_leaf < 3.0) & (cnt > 3.0)

            @pl.when(apply_update)
            def _():
                s = jnp.dot(w, out_ref[...],
                            preferred_element_type=jnp.float32)
                avg = s / jnp.maximum(cnt, 1.0)

                dxc = pts_col_ref[:, 0:1] - cx
                dyc = pts_col_ref[:, 1:2] - cy
                dzc = pts_col_ref[:, 2:3] - cz
                dc = jnp.sqrt(dxc * dxc + dyc * dyc + dzc * dzc)
                ball = dc <= _R
                out_ref[...] = jnp.where(ball, avg, out_ref[...])

        return carry

    jax.lax.fori_loop(0, C_MAX, cand_body, 0, unroll=False)


def _build(B, N, D, interpret=False):
    c_max = N // 10 + 4
    return pl.pallas_call(
        _lcn_body,
        grid=(B,),
        in_specs=[
            pl.BlockSpec((None, 3, N), lambda b: (b, 0, 0)),
            pl.BlockSpec((None, N, 3), lambda b: (b, 0, 0)),
            pl.BlockSpec((None, 3, N), lambda b: (b, 0, 0),
                         memory_space=pltpu.SMEM),
            pl.BlockSpec((None, 1, N), lambda b: (b, 0, 0)),
            pl.BlockSpec((None, 1, N), lambda b: (b, 0, 0),
                         memory_space=pltpu.SMEM),
            pl.BlockSpec((None, N, D), lambda b: (b, 0, 0)),
        ],
        out_specs=pl.BlockSpec((None, N, D), lambda b: (b, 0, 0)),
        out_shape=jax.ShapeDtypeStruct((B, N, D), jnp.float32),
        scratch_shapes=[pltpu.SMEM((1, c_max), jnp.int32)],
        interpret=interpret,
    )


@jax.jit
def kernel(points, features, leaf_mask):
    B, N, D = features.shape
    pts_row = jnp.swapaxes(points, 1, 2)
    lmf = leaf_mask.astype(jnp.float32)[:, None, :]
    lmi = leaf_mask.astype(jnp.int32)[:, None, :]
    call = _build(B, N, D)
    return call(pts_row, points, pts_row, lmf, lmi, features)
